# baseline (speedup 1.0000x reference)
import functools

import jax
import jax.numpy as jnp
from jax import lax
from jax.experimental import pallas as pl
from jax.experimental.pallas import tpu as pltpu
from jax.experimental.pallas import tpu_sc as plsc

_N = 10000
_E = 160000
_T = 4
_IN_C = 32
_OUT_C = 32
_D_EDGE = 16
_HID = 32

_C4 = _T * 32
_NW = 32
_CH = 40
_EA = 81920
_EBH = _E - _EA
_ZR = 624
_ZR_TAIL = _N - 16 * _ZR

_EB = 1280
_NB = 2000


def _sc_mesh():
    return plsc.VectorSubcoreMesh(core_axis_name="c", subcore_axis_name="s")


def _gather(xcat, idx3d, nch):
    pw = nch * _CH

    @functools.partial(
        pl.kernel,
        out_type=jax.ShapeDtypeStruct((_NW * pw, _C4), jnp.float32),
        mesh=_sc_mesh(),
        scratch_types=[
            pltpu.VMEM((nch, _CH), jnp.int32),
            pltpu.VMEM((_CH, _C4), jnp.float32),
            pltpu.SemaphoreType.DMA,
        ],
    )
    def k(x_hbm, idx_hbm, out_hbm, idx_v, rows_v, sem):
        wid = lax.axis_index("s") * 2 + lax.axis_index("c")
        pltpu.sync_copy(idx_hbm.at[wid], idx_v)

        def body(j, carry):
            off = wid * pw + j * _CH
            pltpu.async_copy(x_hbm.at[idx_v.at[j]], rows_v, sem).wait()
            pltpu.sync_copy(rows_v, out_hbm.at[pl.ds(off, _CH)])
            return carry

        lax.fori_loop(0, nch, body, 0)

    return k(xcat, idx3d)


def _scatter(msg4, idx3d, init, nch):
    pw = nch * _CH

    @functools.partial(
        pl.kernel,
        out_type=jax.ShapeDtypeStruct((2, _N, _C4), jnp.float32),
        mesh=_sc_mesh(),
        scratch_types=[
            pltpu.VMEM((nch, _CH), jnp.int32),
            pltpu.VMEM((_CH, _C4), jnp.float32),
            pltpu.VMEM_SHARED((_N, _C4), jnp.float32),
            pltpu.SemaphoreType.DMA,
        ],
    )
    def k(msg_hbm, idx_hbm, init_hbm, out_hbm, idx_v, rows_v, agg_sh, sem):
        cid = lax.axis_index("c")
        sid = lax.axis_index("s")
        wid = sid * 2 + cid
        pltpu.sync_copy(init_hbm.at[cid, pl.ds(sid * _ZR, _ZR)],
                        agg_sh.at[pl.ds(sid * _ZR, _ZR)])

        @pl.when(sid == 15)
        def _():
            pltpu.sync_copy(init_hbm.at[cid, pl.ds(16 * _ZR, _ZR_TAIL)],
                            agg_sh.at[pl.ds(16 * _ZR, _ZR_TAIL)])

        pltpu.sync_copy(idx_hbm.at[wid], idx_v)
        plsc.subcore_barrier()

        def body(j, carry):
            off = wid * pw + j * _CH
            pltpu.sync_copy(msg_hbm.at[pl.ds(off, _CH)], rows_v)
            pltpu.sync_copy(rows_v, agg_sh.at[idx_v.at[j]], add=True)
            return carry

        lax.fori_loop(0, nch, body, 0)
        plsc.subcore_barrier()
        pltpu.sync_copy(
            agg_sh.at[pl.ds(sid * _ZR, _ZR)],
            out_hbm.at[cid, pl.ds(sid * _ZR, _ZR)],
        )

        @pl.when(sid == 15)
        def _():
            pltpu.sync_copy(agg_sh.at[pl.ds(16 * _ZR, _ZR_TAIL)],
                            out_hbm.at[cid, pl.ds(16 * _ZR, _ZR_TAIL)])

    return k(msg4, idx3d, init)


_ZW = _D_EDGE * 32


def _mm(a, b, out=jnp.float32):
    return lax.dot_general(a, b, (((1,), (0,)), ((), ())),
                           preferred_element_type=out)


def _mm_t(a, b, out=jnp.float32):
    return lax.dot_general(a, b, (((0,), (0,)), ((), ())),
                           preferred_element_type=out)


def _dense_body(eat_ref, xj_ref, rrep_ref, srep_ref, w24_ref, bbig_ref,
                msg_ref):
    bf = jnp.bfloat16
    xj4 = xj_ref[...].astype(bf)
    eat = eat_ref[...].astype(bf)
    rrep = rrep_ref[...]
    acc = _mm(xj4, bbig_ref[...])
    for t in range(_T):
        eat_t = eat[t * _D_EDGE : (t + 1) * _D_EDGE]
        er = _mm_t(eat_t, rrep)
        xr = _mm(xj4, srep_ref[t])
        acc = acc + _mm(er * xr, w24_ref[t])
    msg_ref[...] = acc


def _dense(eat64, xj4, rrep, srep, w24, bbig, e_h, blk_off):
    grid = (e_h // _EB,)
    return pl.pallas_call(
        _dense_body,
        grid=grid,
        in_specs=[
            pl.BlockSpec((_T * _D_EDGE, _EB), lambda i: (0, i + blk_off)),
            pl.BlockSpec((_EB, _C4), lambda i: (i, 0)),
            pl.BlockSpec((_D_EDGE, _ZW), lambda i: (0, 0)),
            pl.BlockSpec((_T, _C4, _ZW), lambda i: (0, 0, 0)),
            pl.BlockSpec((_T, _ZW, _C4), lambda i: (0, 0, 0)),
            pl.BlockSpec((_C4, _C4), lambda i: (0, 0)),
        ],
        out_specs=pl.BlockSpec((_EB, _C4), lambda i: (i, 0)),
        out_shape=jax.ShapeDtypeStruct((e_h, _C4), jnp.float32),
    )(eat64, xj4, rrep, srep, w24, bbig)


def _head_body(p_ref, x_ref, wr_ref, wih_ref, whh_ref, brow_ref, bsum_ref,
               h_ref, c_ref):
    wr = wr_ref[...]
    wih = wih_ref[...]
    whh = whh_ref[...]
    brow = brow_ref[...]
    bsum = bsum_ref[...]
    h = jnp.zeros((_NB, _HID), jnp.float32)
    c = jnp.zeros((_NB, _HID), jnp.float32)
    for t in range(_T):
        sl = slice(t * 32, (t + 1) * 32)
        xt = x_ref[:, sl]
        agg = p_ref[0][:, sl] + p_ref[1][:, sl]
        s = jax.nn.relu(
            agg
            + lax.dot_general(xt, wr, (((1,), (0,)), ((), ())),
                              preferred_element_type=jnp.float32)
            + brow
        )
        g = (
            lax.dot_general(s, wih, (((1,), (0,)), ((), ())),
                            preferred_element_type=jnp.float32)
            + lax.dot_general(h, whh, (((1,), (0,)), ((), ())),
                              preferred_element_type=jnp.float32)
            + bsum
        )
        i_g = jax.nn.sigmoid(g[:, 0:32])
        f_g = jax.nn.sigmoid(g[:, 32:64])
        g_g = jnp.tanh(g[:, 64:96])
        o_g = jax.nn.sigmoid(g[:, 96:128])
        c = f_g * c + i_g * g_g
        h = o_g * jnp.tanh(c)
    h_ref[...] = h
    c_ref[...] = c


def _head(partials, xcat, wr_t, wih_t, whh_t, brow, bsum):
    grid = (_N // _NB,)
    return pl.pallas_call(
        _head_body,
        grid=grid,
        in_specs=[
            pl.BlockSpec((2, _NB, _C4), lambda i: (0, i, 0)),
            pl.BlockSpec((_NB, _C4), lambda i: (i, 0)),
            pl.BlockSpec((32, 32), lambda i: (0, 0)),
            pl.BlockSpec((32, 128), lambda i: (0, 0)),
            pl.BlockSpec((_HID, 128), lambda i: (0, 0)),
            pl.BlockSpec((1, 32), lambda i: (0, 0)),
            pl.BlockSpec((1, 128), lambda i: (0, 0)),
        ],
        out_specs=[
            pl.BlockSpec((_NB, _HID), lambda i: (i, 0)),
            pl.BlockSpec((_NB, _HID), lambda i: (i, 0)),
        ],
        out_shape=[
            jax.ShapeDtypeStruct((_N, _HID), jnp.float32),
            jax.ShapeDtypeStruct((_N, _HID), jnp.float32),
        ],
    )(partials, xcat, wr_t, wih_t, whh_t, brow, bsum)


def kernel(x, edge_index, edge_attr, W_lin, b_lin, W_root, bias, W_ih, W_hh,
           b_ih, b_hh):
    src = edge_index[0]
    dst = edge_index[1]
    ncha = _EA // _NW // _CH
    nchb = _EBH // _NW // _CH
    src3a = src[:_EA].reshape(_NW, ncha, _CH)
    src3b = src[_EA:].reshape(_NW, nchb, _CH)
    dst3a = dst[:_EA].reshape(_NW, ncha, _CH)
    dst3b = dst[_EA:].reshape(_NW, nchb, _CH)

    xcat = x.transpose(1, 0, 2).reshape(_N, _C4)

    w3 = W_lin.reshape(_IN_C, _OUT_C, _D_EDGE)
    w2 = jnp.concatenate(
        [w3.transpose(2, 0, 1).reshape(_D_EDGE * _IN_C, _OUT_C),
         b_lin.reshape(_IN_C, _OUT_C)],
        axis=0,
    )
    w2a = w2[: _ZW]
    rrep = jnp.repeat(jnp.eye(_D_EDGE, dtype=jnp.float32), 32, axis=1)
    eye32 = jnp.eye(32, dtype=jnp.float32)
    srep = jnp.stack([
        jnp.tile(jnp.pad(eye32, ((t * 32, 96 - t * 32), (0, 0))), (1, 16))
        for t in range(_T)])
    w24 = jnp.stack([
        jnp.pad(w2a, ((0, 0), (t * 32, 96 - t * 32))) for t in range(_T)])
    bbig = jnp.kron(jnp.eye(_T, dtype=jnp.float32),
                    b_lin.reshape(_IN_C, _OUT_C))
    bf = jnp.bfloat16
    rrep, srep, bbig = (a.astype(bf) for a in (rrep, srep, bbig))

    eat64 = edge_attr.transpose(0, 2, 1).reshape(_T * _D_EDGE, _E)

    xj4a = _gather(xcat, src3a, ncha)
    xj4b = _gather(xcat, src3b, nchb)
    msg4a = _dense(eat64, xj4a, rrep, srep, w24, bbig, _EA, 0)
    msg4b = _dense(eat64, xj4b, rrep, srep, w24, bbig, _EBH, _EA // _EB)
    pzero = jnp.zeros((2, _N, _C4), jnp.float32)
    pa = _scatter(msg4a, dst3a, pzero, ncha)
    partials = _scatter(msg4b, dst3b, pa, nchb)

    h_n, c_n = _head(
        partials, xcat,
        W_root.T, W_ih.T, W_hh.T,
        bias.reshape(1, 32), (b_ih + b_hh).reshape(1, 128),
    )
    return (h_n[None], c_n[None])


# speedup vs baseline: 1.0814x; 1.0814x over previous
import functools

import jax
import jax.numpy as jnp
from jax import lax
from jax.experimental import pallas as pl
from jax.experimental.pallas import tpu as pltpu
from jax.experimental.pallas import tpu_sc as plsc

_N = 10000
_E = 160000
_T = 4
_IN_C = 32
_OUT_C = 32
_D_EDGE = 16
_HID = 32

_C4 = _T * 32
_NW = 32
_CH = 40
_EA = 81920
_EBH = _E - _EA
_ZR = 624
_ZR_TAIL = _N - 16 * _ZR

_EB = 1280
_NB = 2000


def _sc_mesh():
    return plsc.VectorSubcoreMesh(core_axis_name="c", subcore_axis_name="s")


def _gather(xcat, idx3d, nch):
    pw = nch * _CH

    @functools.partial(
        pl.kernel,
        out_type=jax.ShapeDtypeStruct((_NW * pw, _C4), jnp.float32),
        mesh=_sc_mesh(),
        scratch_types=[
            pltpu.VMEM((nch, _CH), jnp.int32),
            pltpu.VMEM((_CH, _C4), jnp.float32),
            pltpu.SemaphoreType.DMA,
        ],
    )
    def k(x_hbm, idx_hbm, out_hbm, idx_v, rows_v, sem):
        wid = lax.axis_index("s") * 2 + lax.axis_index("c")
        pltpu.sync_copy(idx_hbm.at[wid], idx_v)

        def body(j, carry):
            off = wid * pw + j * _CH
            pltpu.async_copy(x_hbm.at[idx_v.at[j]], rows_v, sem).wait()
            pltpu.sync_copy(rows_v, out_hbm.at[pl.ds(off, _CH)])
            return carry

        lax.fori_loop(0, nch, body, 0)

    return k(xcat, idx3d)


def _scatter(msg4, idx3d, init, nch):
    pw = nch * _CH

    @functools.partial(
        pl.kernel,
        out_type=jax.ShapeDtypeStruct((2, _N, _C4), jnp.float32),
        mesh=_sc_mesh(),
        scratch_types=[
            pltpu.VMEM((nch, _CH), jnp.int32),
            pltpu.VMEM((_CH, _C4), jnp.float32),
            pltpu.VMEM_SHARED((_N, _C4), jnp.float32),
            pltpu.SemaphoreType.DMA,
        ],
    )
    def k(msg_hbm, idx_hbm, init_hbm, out_hbm, idx_v, rows_v, agg_sh, sem):
        cid = lax.axis_index("c")
        sid = lax.axis_index("s")
        wid = sid * 2 + cid
        pltpu.sync_copy(init_hbm.at[cid, pl.ds(sid * _ZR, _ZR)],
                        agg_sh.at[pl.ds(sid * _ZR, _ZR)])

        @pl.when(sid == 15)
        def _():
            pltpu.sync_copy(init_hbm.at[cid, pl.ds(16 * _ZR, _ZR_TAIL)],
                            agg_sh.at[pl.ds(16 * _ZR, _ZR_TAIL)])

        pltpu.sync_copy(idx_hbm.at[wid], idx_v)
        plsc.subcore_barrier()

        def body(j, carry):
            off = wid * pw + j * _CH
            pltpu.sync_copy(msg_hbm.at[pl.ds(off, _CH)], rows_v)
            pltpu.sync_copy(rows_v, agg_sh.at[idx_v.at[j]], add=True)
            return carry

        lax.fori_loop(0, nch, body, 0)
        plsc.subcore_barrier()
        pltpu.sync_copy(
            agg_sh.at[pl.ds(sid * _ZR, _ZR)],
            out_hbm.at[cid, pl.ds(sid * _ZR, _ZR)],
        )

        @pl.when(sid == 15)
        def _():
            pltpu.sync_copy(agg_sh.at[pl.ds(16 * _ZR, _ZR_TAIL)],
                            out_hbm.at[cid, pl.ds(16 * _ZR, _ZR_TAIL)])

    return k(msg4, idx3d, init)


_ZW = _D_EDGE * 32


def _mm(a, b, out=jnp.float32):
    return lax.dot_general(a, b, (((1,), (0,)), ((), ())),
                           preferred_element_type=out)


def _mm_t(a, b, out=jnp.float32):
    return lax.dot_general(a, b, (((0,), (0,)), ((), ())),
                           preferred_element_type=out)


def _dense_body(eat_ref, xj_ref, rrep_ref, srep_ref, w24_ref, bbig_ref,
                msg_ref):
    xj4 = xj_ref[...]
    eat = eat_ref[...]
    rrep = rrep_ref[...]
    acc = _mm(xj4, bbig_ref[...])
    for t in range(_T):
        eat_t = eat[t * _D_EDGE : (t + 1) * _D_EDGE]
        er = _mm_t(eat_t, rrep)
        xr = _mm(xj4, srep_ref[t])
        acc = acc + _mm(er * xr, w24_ref[t])
    msg_ref[...] = acc


def _dense(eat64, xj4, rrep, srep, w24, bbig, e_h, blk_off):
    grid = (e_h // _EB,)
    return pl.pallas_call(
        _dense_body,
        grid=grid,
        in_specs=[
            pl.BlockSpec((_T * _D_EDGE, _EB), lambda i: (0, i + blk_off)),
            pl.BlockSpec((_EB, _C4), lambda i: (i, 0)),
            pl.BlockSpec((_D_EDGE, _ZW), lambda i: (0, 0)),
            pl.BlockSpec((_T, _C4, _ZW), lambda i: (0, 0, 0)),
            pl.BlockSpec((_T, _ZW, _C4), lambda i: (0, 0, 0)),
            pl.BlockSpec((_C4, _C4), lambda i: (0, 0)),
        ],
        out_specs=pl.BlockSpec((_EB, _C4), lambda i: (i, 0)),
        out_shape=jax.ShapeDtypeStruct((e_h, _C4), jnp.float32),
    )(eat64, xj4, rrep, srep, w24, bbig)


def _head_body(p_ref, x_ref, wr_ref, wih_ref, whh_ref, brow_ref, bsum_ref,
               h_ref, c_ref):
    wr = wr_ref[...]
    wih = wih_ref[...]
    whh = whh_ref[...]
    brow = brow_ref[...]
    bsum = bsum_ref[...]
    h = jnp.zeros((_NB, _HID), jnp.float32)
    c = jnp.zeros((_NB, _HID), jnp.float32)
    for t in range(_T):
        sl = slice(t * 32, (t + 1) * 32)
        xt = x_ref[:, sl]
        agg = p_ref[0][:, sl] + p_ref[1][:, sl]
        s = jax.nn.relu(
            agg
            + lax.dot_general(xt, wr, (((1,), (0,)), ((), ())),
                              preferred_element_type=jnp.float32)
            + brow
        )
        g = (
            lax.dot_general(s, wih, (((1,), (0,)), ((), ())),
                            preferred_element_type=jnp.float32)
            + lax.dot_general(h, whh, (((1,), (0,)), ((), ())),
                              preferred_element_type=jnp.float32)
            + bsum
        )
        i_g = jax.nn.sigmoid(g[:, 0:32])
        f_g = jax.nn.sigmoid(g[:, 32:64])
        g_g = jnp.tanh(g[:, 64:96])
        o_g = jax.nn.sigmoid(g[:, 96:128])
        c = f_g * c + i_g * g_g
        h = o_g * jnp.tanh(c)
    h_ref[...] = h
    c_ref[...] = c


def _head(partials, xcat, wr_t, wih_t, whh_t, brow, bsum):
    grid = (_N // _NB,)
    return pl.pallas_call(
        _head_body,
        grid=grid,
        in_specs=[
            pl.BlockSpec((2, _NB, _C4), lambda i: (0, i, 0)),
            pl.BlockSpec((_NB, _C4), lambda i: (i, 0)),
            pl.BlockSpec((32, 32), lambda i: (0, 0)),
            pl.BlockSpec((32, 128), lambda i: (0, 0)),
            pl.BlockSpec((_HID, 128), lambda i: (0, 0)),
            pl.BlockSpec((1, 32), lambda i: (0, 0)),
            pl.BlockSpec((1, 128), lambda i: (0, 0)),
        ],
        out_specs=[
            pl.BlockSpec((_NB, _HID), lambda i: (i, 0)),
            pl.BlockSpec((_NB, _HID), lambda i: (i, 0)),
        ],
        out_shape=[
            jax.ShapeDtypeStruct((_N, _HID), jnp.float32),
            jax.ShapeDtypeStruct((_N, _HID), jnp.float32),
        ],
    )(partials, xcat, wr_t, wih_t, whh_t, brow, bsum)


def kernel(x, edge_index, edge_attr, W_lin, b_lin, W_root, bias, W_ih, W_hh,
           b_ih, b_hh):
    src = edge_index[0]
    dst = edge_index[1]
    ofs = [0, 40960, 80640, 120320, _E]

    xcat = x.transpose(1, 0, 2).reshape(_N, _C4)

    w3 = W_lin.reshape(_IN_C, _OUT_C, _D_EDGE)
    w2 = jnp.concatenate(
        [w3.transpose(2, 0, 1).reshape(_D_EDGE * _IN_C, _OUT_C),
         b_lin.reshape(_IN_C, _OUT_C)],
        axis=0,
    )
    w2a = w2[: _ZW]
    rrep = jnp.repeat(jnp.eye(_D_EDGE, dtype=jnp.float32), 32, axis=1)
    eye32 = jnp.eye(32, dtype=jnp.float32)
    srep = jnp.stack([
        jnp.tile(jnp.pad(eye32, ((t * 32, 96 - t * 32), (0, 0))), (1, 16))
        for t in range(_T)])
    w24 = jnp.stack([
        jnp.pad(w2a, ((0, 0), (t * 32, 96 - t * 32))) for t in range(_T)])
    bbig = jnp.kron(jnp.eye(_T, dtype=jnp.float32),
                    b_lin.reshape(_IN_C, _OUT_C))

    eat64 = edge_attr.transpose(0, 2, 1).reshape(_T * _D_EDGE, _E)

    msgs, dsts, nchs = [], [], []
    for lo, hi in zip(ofs[:-1], ofs[1:]):
        e_h = hi - lo
        nch = e_h // _NW // _CH
        src3 = src[lo:hi].reshape(_NW, nch, _CH)
        xj4 = _gather(xcat, src3, nch)
        msgs.append(_dense(eat64, xj4, rrep, srep, w24, bbig, e_h, lo // _EB))
        dsts.append(dst[lo:hi].reshape(_NW, nch, _CH))
        nchs.append(nch)
    partials = jnp.zeros((2, _N, _C4), jnp.float32)
    for msg4, dst3, nch in zip(msgs, dsts, nchs):
        partials = _scatter(msg4, dst3, partials, nch)

    h_n, c_n = _head(
        partials, xcat,
        W_root.T, W_ih.T, W_hh.T,
        bias.reshape(1, 32), (b_ih + b_hh).reshape(1, 128),
    )
    return (h_n[None], c_n[None])


# speedup vs baseline: 1.1497x; 1.0632x over previous
import functools

import jax
import jax.numpy as jnp
from jax import lax
from jax.experimental import pallas as pl
from jax.experimental.pallas import tpu as pltpu
from jax.experimental.pallas import tpu_sc as plsc

_N = 10000
_E = 160000
_T = 4
_IN_C = 32
_OUT_C = 32
_D_EDGE = 16
_HID = 32

_C4 = _T * 32
_NW = 32
_CH = 40
_EA = 81920
_EBH = _E - _EA
_ZR = 624
_ZR_TAIL = _N - 16 * _ZR

_EB = 1280
_NB = 2000


def _sc_mesh():
    return plsc.VectorSubcoreMesh(core_axis_name="c", subcore_axis_name="s")


def _gather(xcat, idx3d, nch, ch):
    pw = nch * ch

    @functools.partial(
        pl.kernel,
        out_type=jax.ShapeDtypeStruct((_NW * pw, _C4), jnp.float32),
        mesh=_sc_mesh(),
        scratch_types=[
            pltpu.VMEM((nch, ch), jnp.int32),
            pltpu.VMEM((ch, _C4), jnp.float32),
            pltpu.SemaphoreType.DMA,
        ],
    )
    def k(x_hbm, idx_hbm, out_hbm, idx_v, rows_v, sem):
        wid = lax.axis_index("s") * 2 + lax.axis_index("c")
        pltpu.sync_copy(idx_hbm.at[wid], idx_v)

        def body(j, carry):
            off = wid * pw + j * ch
            pltpu.async_copy(x_hbm.at[idx_v.at[j]], rows_v, sem).wait()
            pltpu.sync_copy(rows_v, out_hbm.at[pl.ds(off, ch)])
            return carry

        lax.fori_loop(0, nch, body, 0)

    return k(xcat, idx3d)


def _scatter(msg4, idx3d, init, nch, ch):
    pw = nch * ch

    @functools.partial(
        pl.kernel,
        out_type=jax.ShapeDtypeStruct((2, _N, _C4), jnp.float32),
        mesh=_sc_mesh(),
        scratch_types=[
            pltpu.VMEM((nch, ch), jnp.int32),
            pltpu.VMEM((ch, _C4), jnp.float32),
            pltpu.VMEM_SHARED((_N, _C4), jnp.float32),
            pltpu.SemaphoreType.DMA,
        ],
    )
    def k(msg_hbm, idx_hbm, init_hbm, out_hbm, idx_v, rows_v, agg_sh, sem):
        cid = lax.axis_index("c")
        sid = lax.axis_index("s")
        wid = sid * 2 + cid
        pltpu.sync_copy(init_hbm.at[cid, pl.ds(sid * _ZR, _ZR)],
                        agg_sh.at[pl.ds(sid * _ZR, _ZR)])

        @pl.when(sid == 15)
        def _():
            pltpu.sync_copy(init_hbm.at[cid, pl.ds(16 * _ZR, _ZR_TAIL)],
                            agg_sh.at[pl.ds(16 * _ZR, _ZR_TAIL)])

        pltpu.sync_copy(idx_hbm.at[wid], idx_v)
        plsc.subcore_barrier()

        def body(j, carry):
            off = wid * pw + j * ch
            pltpu.sync_copy(msg_hbm.at[pl.ds(off, ch)], rows_v)
            pltpu.sync_copy(rows_v, agg_sh.at[idx_v.at[j]], add=True)
            return carry

        lax.fori_loop(0, nch, body, 0)
        plsc.subcore_barrier()
        pltpu.sync_copy(
            agg_sh.at[pl.ds(sid * _ZR, _ZR)],
            out_hbm.at[cid, pl.ds(sid * _ZR, _ZR)],
        )

        @pl.when(sid == 15)
        def _():
            pltpu.sync_copy(agg_sh.at[pl.ds(16 * _ZR, _ZR_TAIL)],
                            out_hbm.at[cid, pl.ds(16 * _ZR, _ZR_TAIL)])

    return k(msg4, idx3d, init)


_ZW = _D_EDGE * 32


def _mm(a, b, out=jnp.float32):
    return lax.dot_general(a, b, (((1,), (0,)), ((), ())),
                           preferred_element_type=out)


def _mm_t(a, b, out=jnp.float32):
    return lax.dot_general(a, b, (((0,), (0,)), ((), ())),
                           preferred_element_type=out)


def _dense_body(eat_ref, xj_ref, rrep_ref, srep_ref, w24_ref, bbig_ref,
                msg_ref):
    xj4 = xj_ref[...]
    eat = eat_ref[...]
    rrep = rrep_ref[...]
    acc = _mm(xj4, bbig_ref[...])
    for t in range(_T):
        eat_t = eat[t * _D_EDGE : (t + 1) * _D_EDGE]
        er = _mm_t(eat_t, rrep)
        xr = _mm(xj4, srep_ref[t])
        acc = acc + _mm(er * xr, w24_ref[t])
    msg_ref[...] = acc


def _dense(eat64, xj4, rrep, srep, w24, bbig, e_h, blk_off):
    grid = (e_h // _EB,)
    return pl.pallas_call(
        _dense_body,
        grid=grid,
        in_specs=[
            pl.BlockSpec((_T * _D_EDGE, _EB), lambda i: (0, i + blk_off)),
            pl.BlockSpec((_EB, _C4), lambda i: (i, 0)),
            pl.BlockSpec((_D_EDGE, _ZW), lambda i: (0, 0)),
            pl.BlockSpec((_T, _C4, _ZW), lambda i: (0, 0, 0)),
            pl.BlockSpec((_T, _ZW, _C4), lambda i: (0, 0, 0)),
            pl.BlockSpec((_C4, _C4), lambda i: (0, 0)),
        ],
        out_specs=pl.BlockSpec((_EB, _C4), lambda i: (i, 0)),
        out_shape=jax.ShapeDtypeStruct((e_h, _C4), jnp.float32),
    )(eat64, xj4, rrep, srep, w24, bbig)


def _head_body(p_ref, x_ref, wr_ref, wih_ref, whh_ref, brow_ref, bsum_ref,
               h_ref, c_ref):
    wr = wr_ref[...]
    wih = wih_ref[...]
    whh = whh_ref[...]
    brow = brow_ref[...]
    bsum = bsum_ref[...]
    h = jnp.zeros((_NB, _HID), jnp.float32)
    c = jnp.zeros((_NB, _HID), jnp.float32)
    for t in range(_T):
        sl = slice(t * 32, (t + 1) * 32)
        xt = x_ref[:, sl]
        agg = p_ref[0][:, sl] + p_ref[1][:, sl]
        s = jax.nn.relu(
            agg
            + lax.dot_general(xt, wr, (((1,), (0,)), ((), ())),
                              preferred_element_type=jnp.float32)
            + brow
        )
        g = (
            lax.dot_general(s, wih, (((1,), (0,)), ((), ())),
                            preferred_element_type=jnp.float32)
            + lax.dot_general(h, whh, (((1,), (0,)), ((), ())),
                              preferred_element_type=jnp.float32)
            + bsum
        )
        i_g = jax.nn.sigmoid(g[:, 0:32])
        f_g = jax.nn.sigmoid(g[:, 32:64])
        g_g = jnp.tanh(g[:, 64:96])
        o_g = jax.nn.sigmoid(g[:, 96:128])
        c = f_g * c + i_g * g_g
        h = o_g * jnp.tanh(c)
    h_ref[...] = h
    c_ref[...] = c


def _head(partials, xcat, wr_t, wih_t, whh_t, brow, bsum):
    grid = (_N // _NB,)
    return pl.pallas_call(
        _head_body,
        grid=grid,
        in_specs=[
            pl.BlockSpec((2, _NB, _C4), lambda i: (0, i, 0)),
            pl.BlockSpec((_NB, _C4), lambda i: (i, 0)),
            pl.BlockSpec((32, 32), lambda i: (0, 0)),
            pl.BlockSpec((32, 128), lambda i: (0, 0)),
            pl.BlockSpec((_HID, 128), lambda i: (0, 0)),
            pl.BlockSpec((1, 32), lambda i: (0, 0)),
            pl.BlockSpec((1, 128), lambda i: (0, 0)),
        ],
        out_specs=[
            pl.BlockSpec((_NB, _HID), lambda i: (i, 0)),
            pl.BlockSpec((_NB, _HID), lambda i: (i, 0)),
        ],
        out_shape=[
            jax.ShapeDtypeStruct((_N, _HID), jnp.float32),
            jax.ShapeDtypeStruct((_N, _HID), jnp.float32),
        ],
    )(partials, xcat, wr_t, wih_t, whh_t, brow, bsum)


def kernel(x, edge_index, edge_attr, W_lin, b_lin, W_root, bias, W_ih, W_hh,
           b_ih, b_hh):
    src = edge_index[0]
    dst = edge_index[1]
    ofs = [0, 20480, 61440, 102400, 143360, _E]
    chs = [128, 128, 128, 128, 40]

    xcat = x.transpose(1, 0, 2).reshape(_N, _C4)

    w3 = W_lin.reshape(_IN_C, _OUT_C, _D_EDGE)
    w2 = jnp.concatenate(
        [w3.transpose(2, 0, 1).reshape(_D_EDGE * _IN_C, _OUT_C),
         b_lin.reshape(_IN_C, _OUT_C)],
        axis=0,
    )
    w2a = w2[: _ZW]
    rrep = jnp.repeat(jnp.eye(_D_EDGE, dtype=jnp.float32), 32, axis=1)
    eye32 = jnp.eye(32, dtype=jnp.float32)
    srep = jnp.stack([
        jnp.tile(jnp.pad(eye32, ((t * 32, 96 - t * 32), (0, 0))), (1, 16))
        for t in range(_T)])
    w24 = jnp.stack([
        jnp.pad(w2a, ((0, 0), (t * 32, 96 - t * 32))) for t in range(_T)])
    bbig = jnp.kron(jnp.eye(_T, dtype=jnp.float32),
                    b_lin.reshape(_IN_C, _OUT_C))

    eat64 = edge_attr.transpose(0, 2, 1).reshape(_T * _D_EDGE, _E)

    msgs, dsts, nchs = [], [], []
    for lo, hi, ch in zip(ofs[:-1], ofs[1:], chs):
        e_h = hi - lo
        nch = e_h // _NW // ch
        src3 = src[lo:hi].reshape(_NW, nch, ch)
        xj4 = _gather(xcat, src3, nch, ch)
        msgs.append(_dense(eat64, xj4, rrep, srep, w24, bbig, e_h, lo // _EB))
        dsts.append(dst[lo:hi].reshape(_NW, nch, ch))
        nchs.append((nch, ch))
    partials = jnp.zeros((2, _N, _C4), jnp.float32)
    for msg4, dst3, (nch, ch) in zip(msgs, dsts, nchs):
        partials = _scatter(msg4, dst3, partials, nch, ch)

    h_n, c_n = _head(
        partials, xcat,
        W_root.T, W_ih.T, W_hh.T,
        bias.reshape(1, 32), (b_ih + b_hh).reshape(1, 128),
    )
    return (h_n[None], c_n[None])
